# baseline (speedup 1.0000x reference)
import functools

import jax
import jax.numpy as jnp
from jax import lax
from jax.experimental import pallas as pl
from jax.experimental.pallas import tpu as pltpu
from jax.experimental.pallas import tpu_sc as plsc

B = 16384
D = 64
L = 16
NC = 2
NS = 16
NW = NC * NS
BPW = B // NW
GROUPS = BPW // L


def _transe_body(e1_hbm, rel_hbm, e2_hbm, ent_hbm, relt_hbm, gam_hbm,
                 out_hbm,
                 e1_v, rel_v, e2_v, head_v, relr_v, tail_v, out_v, gam_v,
                 sem1, sem2, sem3):
    wid = lax.axis_index("s") * NC + lax.axis_index("c")
    base = wid * BPW

    pltpu.sync_copy(e1_hbm.at[pl.ds(base, BPW)], e1_v)
    pltpu.sync_copy(e2_hbm.at[pl.ds(base, BPW)], e2_v)
    pltpu.sync_copy(rel_hbm.at[pl.ds(base, BPW)], rel_v)
    pltpu.sync_copy(gam_hbm, gam_v)

    cp1 = pltpu.async_copy(ent_hbm.at[e1_v], head_v, sem1)
    cp2 = pltpu.async_copy(ent_hbm.at[e2_v], tail_v, sem2)
    cp3 = pltpu.async_copy(relt_hbm.at[rel_v], relr_v, sem3)
    cp1.wait()
    cp2.wait()
    cp3.wait()

    gam = gam_v[...]

    def group_body(g, carry):
        row0 = g * L
        rows = row0 + lax.iota(jnp.int32, L)
        acc = jnp.zeros((L,), jnp.float32)
        for d in range(D):
            dcol = jnp.full((L,), d, jnp.int32)
            h = plsc.load_gather(head_v, [rows, dcol])
            r = plsc.load_gather(relr_v, [rows, dcol])
            t = plsc.load_gather(tail_v, [rows, dcol])
            acc = acc + jnp.abs(h + r - t)
        score = gam - acc
        out_v[pl.ds(row0, L)] = 1.0 / (1.0 + jnp.exp(-score))
        return carry

    lax.fori_loop(0, GROUPS, group_body, 0)

    pltpu.sync_copy(out_v, out_hbm.at[pl.ds(base, BPW)])


@functools.partial(jax.jit, static_argnums=())
def _transe_call(e1_idx, rel_idx, e2_idx, emb_ent_real, emb_rel_real,
                 gam_vec):
    mesh = plsc.VectorSubcoreMesh(core_axis_name="c", subcore_axis_name="s")
    f = pl.kernel(
        _transe_body,
        mesh=mesh,
        compiler_params=pltpu.CompilerParams(
            needs_layout_passes=False, use_tc_tiling_on_sc=False),
        out_type=jax.ShapeDtypeStruct((B,), jnp.float32),
        scratch_types=[
            pltpu.VMEM((BPW,), jnp.int32),
            pltpu.VMEM((BPW,), jnp.int32),
            pltpu.VMEM((BPW,), jnp.int32),
            pltpu.VMEM((BPW, D), jnp.float32),
            pltpu.VMEM((BPW, D), jnp.float32),
            pltpu.VMEM((BPW, D), jnp.float32),
            pltpu.VMEM((BPW,), jnp.float32),
            pltpu.VMEM((L,), jnp.float32),
            pltpu.SemaphoreType.DMA,
            pltpu.SemaphoreType.DMA,
            pltpu.SemaphoreType.DMA,
        ],
    )
    return f(e1_idx, rel_idx, e2_idx, emb_ent_real, emb_rel_real, gam_vec)


def kernel(e1_idx, rel_idx, e2_idx, emb_ent_real, emb_rel_real, gamma):
    e1 = e1_idx.astype(jnp.int32)
    rel = rel_idx.astype(jnp.int32)
    e2 = e2_idx.astype(jnp.int32)
    gam_vec = jnp.full((L,), gamma, jnp.float32)
    return _transe_call(e1, rel, e2, emb_ent_real, emb_rel_real, gam_vec)


# speedup vs baseline: 1.8167x; 1.8167x over previous
import functools

import jax
import jax.numpy as jnp
from jax import lax
from jax.experimental import pallas as pl
from jax.experimental.pallas import tpu as pltpu
from jax.experimental.pallas import tpu_sc as plsc

B = 16384
D = 64
NE = 1000000
NR = 1000
L = 16
NC = 2
NS = 16
NW = NC * NS
BPW = B // NW
C = 16
NCHUNK = BPW // C


def _transe_body(e1t_hbm, e1r_hbm, e2t_hbm, e2r_hbm, rlt_hbm, rlr_hbm,
                 ent_hbm, rel_hbm, gam_hbm,
                 out_hbm,
                 e1t_v, e1r_v, e2t_v, e2r_v, rlt_v, rlr_v,
                 head_v, tail_v, relr_v, out_v, gam_v,
                 sem1, sem2, sem3):
    wid = lax.axis_index("s") * NC + lax.axis_index("c")
    base = wid * BPW

    pltpu.sync_copy(e1t_hbm.at[pl.ds(base, BPW)], e1t_v)
    pltpu.sync_copy(e1r_hbm.at[pl.ds(base, BPW)], e1r_v)
    pltpu.sync_copy(e2t_hbm.at[pl.ds(base, BPW)], e2t_v)
    pltpu.sync_copy(e2r_hbm.at[pl.ds(base, BPW)], e2r_v)
    pltpu.sync_copy(rlt_hbm.at[pl.ds(base, BPW)], rlt_v)
    pltpu.sync_copy(rlr_hbm.at[pl.ds(base, BPW)], rlr_v)
    pltpu.sync_copy(gam_hbm, gam_v)

    gam = gam_v[...]

    def chunk_body(k, carry):
        off = k * C
        t1v = e1t_v[pl.ds(off, L)]
        t2v = e2t_v[pl.ds(off, L)]
        t3v = rlt_v[pl.ds(off, L)]
        copies = []
        for jj in range(C):
            copies.append(pltpu.async_copy(ent_hbm.at[t1v[jj]],
                                           head_v.at[jj], sem1))
            copies.append(pltpu.async_copy(ent_hbm.at[t2v[jj]],
                                           tail_v.at[jj], sem2))
            copies.append(pltpu.async_copy(rel_hbm.at[t3v[jj]],
                                           relr_v.at[jj], sem3))
        for cp in copies:
            cp.wait()

        j = lax.iota(jnp.int32, L)
        r1 = e1r_v[pl.ds(off, L)]
        r2 = e2r_v[pl.ds(off, L)]
        rr = rlr_v[pl.ds(off, L)]
        acc = jnp.zeros((L,), jnp.float32)
        for d in range(D):
            dcol = jnp.full((L,), d, jnp.int32)
            h = plsc.load_gather(head_v, [j, r1, dcol])
            t = plsc.load_gather(tail_v, [j, r2, dcol])
            r = plsc.load_gather(relr_v, [j, rr, dcol])
            acc = acc + jnp.abs(h + r - t)
        score = gam - acc
        out_v[pl.ds(off, L)] = 1.0 / (1.0 + jnp.exp(-score))
        return carry

    lax.fori_loop(0, NCHUNK, chunk_body, 0)

    pltpu.sync_copy(out_v, out_hbm.at[pl.ds(base, BPW)])


@jax.jit
def _transe_call(e1t, e1r, e2t, e2r, rlt, rlr, ent3, rel3, gam_vec):
    mesh = plsc.VectorSubcoreMesh(core_axis_name="c", subcore_axis_name="s")
    f = pl.kernel(
        _transe_body,
        mesh=mesh,
        compiler_params=pltpu.CompilerParams(needs_layout_passes=False),
        out_type=jax.ShapeDtypeStruct((B,), jnp.float32),
        scratch_types=[
            pltpu.VMEM((BPW,), jnp.int32),
            pltpu.VMEM((BPW,), jnp.int32),
            pltpu.VMEM((BPW,), jnp.int32),
            pltpu.VMEM((BPW,), jnp.int32),
            pltpu.VMEM((BPW,), jnp.int32),
            pltpu.VMEM((BPW,), jnp.int32),
            pltpu.VMEM((C, 8, D), jnp.float32),
            pltpu.VMEM((C, 8, D), jnp.float32),
            pltpu.VMEM((C, 8, D), jnp.float32),
            pltpu.VMEM((BPW,), jnp.float32),
            pltpu.VMEM((L,), jnp.float32),
            pltpu.SemaphoreType.DMA,
            pltpu.SemaphoreType.DMA,
            pltpu.SemaphoreType.DMA,
        ],
    )
    return f(e1t, e1r, e2t, e2r, rlt, rlr, ent3, rel3, gam_vec)


def kernel(e1_idx, rel_idx, e2_idx, emb_ent_real, emb_rel_real, gamma):
    e1 = e1_idx.astype(jnp.int32)
    rel = rel_idx.astype(jnp.int32)
    e2 = e2_idx.astype(jnp.int32)
    ent3 = emb_ent_real.reshape(NE // 8, 8, D)
    rel3 = emb_rel_real.reshape(NR // 8, 8, D)
    gam_vec = jnp.full((L,), gamma, jnp.float32)
    return _transe_call(e1 >> 3, e1 & 7, e2 >> 3, e2 & 7, rel >> 3, rel & 7,
                        ent3, rel3, gam_vec)
